# baseline (speedup 1.0000x reference)
import dataclasses
import functools

import jax
import jax.numpy as jnp
from jax import lax
from jax.experimental import pallas as pl
from jax.experimental.pallas import tpu as pltpu
from jax.experimental.pallas import tpu_sc as plsc

H = W = 512
K = S = 16
B = 4
C = 64
BC = B * C
OH = OW = H // K
L = OH * OW
NW = 32
SLABS_PER_W = BC // NW
KPB = 2
NKB = K // KPB
NBLK = SLABS_PER_W * NKB
INROWS = KPB * K
GBI = 8
NG = OH // GBI


def _fold_sc(x):
    mesh = plsc.VectorSubcoreMesh(core_axis_name="c", subcore_axis_name="s")
    cp = pltpu.CompilerParams()
    if "needs_layout_passes" in pltpu.CompilerParams.__dataclass_fields__:
        cp = dataclasses.replace(cp, needs_layout_passes=False)

    @functools.partial(
        pl.kernel,
        compiler_params=cp,
        out_type=jax.ShapeDtypeStruct((BC, OH, K, W), jnp.float32),
        mesh=mesh,
        scratch_types=[
            pltpu.VMEM((INROWS, L), jnp.float32),
            pltpu.VMEM((INROWS, L), jnp.float32),
            pltpu.VMEM((GBI, KPB, W), jnp.float32),
            pltpu.VMEM((GBI, KPB, W), jnp.float32),
            pltpu.SMEM((1,), jnp.int32),
            pltpu.SemaphoreType.DMA,
            pltpu.SemaphoreType.DMA,
            pltpu.SemaphoreType.DMA,
            pltpu.SemaphoreType.DMA,
        ],
    )
    def body(x_hbm, o_hbm, in0, in1, ob0, ob1, zs, si0, si1, so0, so1):
        cid = lax.axis_index("c")
        sid = lax.axis_index("s")
        wid = sid * 2 + cid
        zs[0] = wid * 0
        dz = zs[0]
        iotd = lax.iota(jnp.int32, 16) + dz
        inbufs = (in0, in1)
        obufs = (ob0, ob1)
        isems = (si0, si1)
        osems = (so0, so1)

        def in_src(blk):
            bc = wid * SLABS_PER_W + blk // NKB
            k8 = blk % NKB
            return x_hbm.at[bc, pl.ds(k8 * INROWS, INROWS), :]

        pltpu.async_copy(in_src(0), inbufs[0], isems[0])

        @pl.loop(0, NBLK // 2)
        def _g(g):
            for p in range(2):
                blk = g * 2 + p
                bc = wid * SLABS_PER_W + blk // NKB
                k8 = blk % NKB
                if p == 0:
                    pltpu.async_copy(in_src(blk + 1), inbufs[1], isems[1])
                else:
                    @pl.when(g < NBLK // 2 - 1)
                    def _():
                        pltpu.async_copy(in_src(blk + 1), inbufs[0],
                                         isems[0])
                pltpu.make_async_copy(in_src(blk), inbufs[p],
                                      isems[p]).wait()
                inb = inbufs[p]

                for gr in range(NG):
                    q = gr % 2
                    ob = obufs[q]
                    dst = o_hbm.at[bc, pl.ds(gr * GBI, GBI),
                                   pl.ds(k8 * KPB, KPB), :]
                    if gr >= 2:
                        pltpu.make_async_copy(ob, dst, osems[q]).wait()
                    else:
                        @pl.when(blk > 0)
                        def _():
                            pltpu.make_async_copy(ob, dst, osems[q]).wait()

                    @plsc.parallel_loop(0, OW, 1, unroll=2)
                    def _bj(bj, gr=gr, inb=inb, ob=ob):
                        colrot = (bj + iotd) & (OW - 1)
                        scol = colrot * K + iotd
                        rows0 = iotd
                        rows1 = iotd + K
                        for bi_l in range(GBI):
                            bi_v = jnp.full((16,), bi_l, jnp.int32) + dz
                            gc = colrot + (gr * GBI + bi_l) * OW
                            for ki_l in range(KPB):
                                v = plsc.load_gather(
                                    inb, [rows0 if ki_l == 0 else rows1,
                                          gc])
                                ki_v = jnp.full((16,), ki_l, jnp.int32) + dz
                                plsc.store_scatter(
                                    ob, [bi_v, ki_v, scol], v)

                    pltpu.async_copy(ob, dst, osems[q])

        last_bc = wid * SLABS_PER_W + SLABS_PER_W - 1
        for gr in (2, 3):
            q = gr % 2
            dst = o_hbm.at[last_bc, pl.ds(gr * GBI, GBI),
                           pl.ds((NKB - 1) * KPB, KPB), :]
            pltpu.make_async_copy(obufs[q], dst, osems[q]).wait()

    return body(x)


def _fold_tc(x):
    n = x.shape[0]

    def tbody(x_ref, o_ref):
        xb = x_ref[0]
        t = xb.reshape(K, K, OH, OW).transpose(2, 0, 3, 1)
        o_ref[0] = t.reshape(OH, K, W)

    return pl.pallas_call(
        tbody,
        grid=(n,),
        in_specs=[pl.BlockSpec((1, K * K, L), lambda i: (i, 0, 0))],
        out_specs=pl.BlockSpec((1, OH, K, W), lambda i: (i, 0, 0, 0)),
        out_shape=jax.ShapeDtypeStruct((n, OH, K, W), jnp.float32),
    )(x)


def kernel(input):
    x = input.reshape(BC, K * K, L)
    out = _fold_tc(x)
    return out.reshape(B, C, H, W)


# speedup vs baseline: 8.4256x; 8.4256x over previous
import dataclasses
import functools

import jax
import jax.numpy as jnp
from jax import lax
from jax.experimental import pallas as pl
from jax.experimental.pallas import tpu as pltpu
from jax.experimental.pallas import tpu_sc as plsc

H = W = 512
K = S = 16
B = 4
C = 64
BC = B * C
OH = OW = H // K
L = OH * OW
NW = 32
SLABS_PER_W = BC // NW
KPB = 2
NKB = K // KPB
NBLK = SLABS_PER_W * NKB
INROWS = KPB * K
GBI = 16
NG = OH // GBI


def _fold_sc(x):
    mesh = plsc.VectorSubcoreMesh(core_axis_name="c", subcore_axis_name="s")
    cp = pltpu.CompilerParams()
    if "needs_layout_passes" in pltpu.CompilerParams.__dataclass_fields__:
        cp = dataclasses.replace(cp, needs_layout_passes=False)

    @functools.partial(
        pl.kernel,
        compiler_params=cp,
        out_type=jax.ShapeDtypeStruct((BC, OH, K, W), jnp.float32),
        mesh=mesh,
        scratch_types=[
            pltpu.VMEM((INROWS, L), jnp.float32),
            pltpu.VMEM((INROWS, L), jnp.float32),
            pltpu.VMEM((GBI, KPB, W), jnp.float32),
            pltpu.VMEM((GBI, KPB, W), jnp.float32),
            pltpu.SMEM((1,), jnp.int32),
            pltpu.SemaphoreType.DMA,
            pltpu.SemaphoreType.DMA,
            pltpu.SemaphoreType.DMA,
            pltpu.SemaphoreType.DMA,
        ],
    )
    def body(x_hbm, o_hbm, in0, in1, ob0, ob1, zs, si0, si1, so0, so1):
        cid = lax.axis_index("c")
        sid = lax.axis_index("s")
        wid = sid * 2 + cid
        zs[0] = wid * 0
        dz = zs[0]
        iotd = lax.iota(jnp.int32, 16) + dz
        inbufs = (in0, in1)
        obufs = (ob0, ob1)
        isems = (si0, si1)
        osems = (so0, so1)

        def in_src(blk):
            bc = wid * SLABS_PER_W + blk // NKB
            k8 = blk % NKB
            return x_hbm.at[bc, pl.ds(k8 * INROWS, INROWS), :]

        pltpu.async_copy(in_src(0), inbufs[0], isems[0])

        @pl.loop(0, NBLK // 2)
        def _g(g):
            for p in range(2):
                blk = g * 2 + p
                bc = wid * SLABS_PER_W + blk // NKB
                k8 = blk % NKB
                if p == 0:
                    pltpu.async_copy(in_src(blk + 1), inbufs[1], isems[1])
                else:
                    @pl.when(g < NBLK // 2 - 1)
                    def _():
                        pltpu.async_copy(in_src(blk + 1), inbufs[0],
                                         isems[0])
                pltpu.make_async_copy(in_src(blk), inbufs[p],
                                      isems[p]).wait()
                inb = inbufs[p]

                for gr in range(NG):
                    q = gr % 2
                    ob = obufs[q]
                    dst = o_hbm.at[bc, pl.ds(gr * GBI, GBI),
                                   pl.ds(k8 * KPB, KPB), :]
                    if gr >= 2:
                        pltpu.make_async_copy(ob, dst, osems[q]).wait()
                    else:
                        @pl.when(blk > 0)
                        def _():
                            pltpu.make_async_copy(ob, dst, osems[q]).wait()

                    @plsc.parallel_loop(0, OW, 1, unroll=2)
                    def _bj(bj, gr=gr, inb=inb, ob=ob):
                        colrot = (bj + iotd) & (OW - 1)
                        scol = colrot * K + iotd
                        rows0 = iotd
                        rows1 = iotd + K
                        for bi_l in range(GBI):
                            bi_v = jnp.full((16,), bi_l, jnp.int32) + dz
                            gc = colrot + (gr * GBI + bi_l) * OW
                            for ki_l in range(KPB):
                                v = plsc.load_gather(
                                    inb, [rows0 if ki_l == 0 else rows1,
                                          gc])
                                ki_v = jnp.full((16,), ki_l, jnp.int32) + dz
                                plsc.store_scatter(
                                    ob, [bi_v, ki_v, scol], v)

                    pltpu.async_copy(ob, dst, osems[q])

        last_bc = wid * SLABS_PER_W + SLABS_PER_W - 1
        for gr in (NG - 2, NG - 1):
            q = gr % 2
            dst = o_hbm.at[last_bc, pl.ds(gr * GBI, GBI),
                           pl.ds((NKB - 1) * KPB, KPB), :]
            pltpu.make_async_copy(obufs[q], dst, osems[q]).wait()

    return body(x)


def kernel(input):
    x = input.reshape(BC, K * K, L)
    out = _fold_sc(x)
    return out.reshape(B, C, H, W)


# speedup vs baseline: 8.4867x; 1.0073x over previous
import dataclasses
import functools

import jax
import jax.numpy as jnp
from jax import lax
from jax.experimental import pallas as pl
from jax.experimental.pallas import tpu as pltpu
from jax.experimental.pallas import tpu_sc as plsc

H = W = 512
K = S = 16
B = 4
C = 64
BC = B * C
OH = OW = H // K
L = OH * OW
NW = 32
SLABS_PER_W = BC // NW
KPB = 2
NKB = K // KPB
NBLK = SLABS_PER_W * NKB
INROWS = KPB * K
GBI = 8
NG = OH // GBI


def _fold_sc(x):
    mesh = plsc.VectorSubcoreMesh(core_axis_name="c", subcore_axis_name="s")
    cp = pltpu.CompilerParams()
    if "needs_layout_passes" in pltpu.CompilerParams.__dataclass_fields__:
        cp = dataclasses.replace(cp, needs_layout_passes=False)

    @functools.partial(
        pl.kernel,
        compiler_params=cp,
        out_type=jax.ShapeDtypeStruct((BC, OH, K, W), jnp.float32),
        mesh=mesh,
        scratch_types=[
            pltpu.VMEM((INROWS, L), jnp.float32),
            pltpu.VMEM((INROWS, L), jnp.float32),
            pltpu.VMEM((GBI, KPB, W), jnp.float32),
            pltpu.VMEM((GBI, KPB, W), jnp.float32),
            pltpu.SMEM((1,), jnp.int32),
            pltpu.SemaphoreType.DMA,
            pltpu.SemaphoreType.DMA,
            pltpu.SemaphoreType.DMA,
            pltpu.SemaphoreType.DMA,
        ],
    )
    def body(x_hbm, o_hbm, in0, in1, ob0, ob1, zs, si0, si1, so0, so1):
        cid = lax.axis_index("c")
        sid = lax.axis_index("s")
        wid = sid * 2 + cid
        zs[0] = wid * 0
        dz = zs[0]
        iotd = lax.iota(jnp.int32, 16) + dz
        inbufs = (in0, in1)
        obufs = (ob0, ob1)
        isems = (si0, si1)
        osems = (so0, so1)

        def in_src(blk):
            bc = wid * SLABS_PER_W + blk // NKB
            k8 = blk % NKB
            return x_hbm.at[bc, pl.ds(k8 * INROWS, INROWS), :]

        def start_in(blk, p):
            bc = wid * SLABS_PER_W + blk // NKB
            k8 = blk % NKB
            hw = INROWS // 2
            src0 = x_hbm.at[bc, pl.ds(k8 * INROWS, hw), :]
            src1 = x_hbm.at[bc, pl.ds(k8 * INROWS + hw, hw), :]
            pltpu.async_copy(src0, inbufs[p].at[pl.ds(0, hw), :], isems[p])
            pltpu.async_copy(src1, inbufs[p].at[pl.ds(hw, hw), :], isems[p])

        start_in(0, 0)

        @pl.loop(0, NBLK // 2)
        def _g(g):
            for p in range(2):
                blk = g * 2 + p
                bc = wid * SLABS_PER_W + blk // NKB
                k8 = blk % NKB
                if p == 0:
                    start_in(blk + 1, 1)
                else:
                    @pl.when(g < NBLK // 2 - 1)
                    def _():
                        start_in(blk + 1, 0)
                pltpu.make_async_copy(in_src(blk), inbufs[p],
                                      isems[p]).wait()
                inb = inbufs[p]

                for gr in range(NG):
                    q = gr % 2
                    ob = obufs[q]
                    dst = o_hbm.at[bc, pl.ds(gr * GBI, GBI),
                                   pl.ds(k8 * KPB, KPB), :]
                    if gr >= 2:
                        pltpu.make_async_copy(ob, dst, osems[q]).wait()
                    else:
                        @pl.when(blk > 0)
                        def _():
                            pltpu.make_async_copy(ob, dst, osems[q]).wait()

                    @plsc.parallel_loop(0, OW, 1, unroll=2)
                    def _bj(bj, gr=gr, inb=inb, ob=ob):
                        colrot = (bj + iotd) & (OW - 1)
                        scol = colrot * K + iotd
                        rows0 = iotd
                        rows1 = iotd + K
                        for bi_l in range(GBI):
                            bi_v = jnp.full((16,), bi_l, jnp.int32) + dz
                            gc = colrot + (gr * GBI + bi_l) * OW
                            for ki_l in range(KPB):
                                v = plsc.load_gather(
                                    inb, [rows0 if ki_l == 0 else rows1,
                                          gc])
                                ki_v = jnp.full((16,), ki_l, jnp.int32) + dz
                                plsc.store_scatter(
                                    ob, [bi_v, ki_v, scol], v)

                    pltpu.async_copy(ob, dst, osems[q])

        last_bc = wid * SLABS_PER_W + SLABS_PER_W - 1
        for gr in (2, 3):
            q = gr % 2
            dst = o_hbm.at[last_bc, pl.ds(gr * GBI, GBI),
                           pl.ds((NKB - 1) * KPB, KPB), :]
            pltpu.make_async_copy(obufs[q], dst, osems[q]).wait()

    return body(x)


def kernel(input):
    x = input.reshape(BC, K * K, L)
    out = _fold_sc(x)
    return out.reshape(B, C, H, W)
